# baseline (speedup 1.0000x reference)
import jax
import jax.numpy as jnp
from jax.experimental import pallas as pl
from jax.experimental.pallas import tpu as pltpu

_DT = 0.05
_G_NA = 120.0
_G_K = 36.0
_G_L = 0.3
_E_NA = 50.0
_E_K = -77.0
_E_L = -54.387

_HI = jax.lax.Precision.HIGHEST


def _dot(a, b, precision=_HI):
    return jax.lax.dot_general(a, b, (((1,), (0,)), ((), ())),
                               precision=precision,
                               preferred_element_type=jnp.float32)


def _split(x):
    hi = x.astype(jnp.bfloat16)
    lo = (x - hi.astype(jnp.float32)).astype(jnp.bfloat16)
    return hi, lo


def _dot3(x, m_hi, m_lo):
    x_hi, x_lo = _split(x)
    d = lambda a, b: _dot(a, b, precision=jax.lax.Precision.DEFAULT)
    return d(x_hi, m_hi) + d(x_hi, m_lo) + d(x_lo, m_hi)


def _hh_body(kt, n_neur, csrc_l, cdst_s, cw_s, gsrc_l, gdst_l, gsrc_s, gdst_s,
             gw_s, iext_ref, vout_ref, V_s, m_s, h_s, n_s,
             mch_s, mcl_s, mgh_s, mgl_s):
    nb = V_s.shape[0]
    nc = csrc_l.shape[1]
    ng = gsrc_l.shape[1]

    @pl.when(pl.program_id(1) == 0)
    def _init():
        f32 = jnp.float32
        ii = jax.lax.broadcasted_iota(jnp.int32, (n_neur, nc), 0)
        A = (ii == csrc_l[...]).astype(f32)
        jj = jax.lax.broadcasted_iota(jnp.int32, (nc, n_neur), 1)
        Wd = jnp.where(jj == cdst_s[...], cw_s[...], 0.0)
        mc = _dot(A, Wd)
        mch_s[...], mcl_s[...] = _split(mc)
        ig = jax.lax.broadcasted_iota(jnp.int32, (n_neur, ng), 0)
        Dg = ((ig == gdst_l[...]).astype(f32)
              - (ig == gsrc_l[...]).astype(f32))
        jg = jax.lax.broadcasted_iota(jnp.int32, (ng, n_neur), 1)
        WDg = (jnp.where(jg == gdst_s[...], gw_s[...], 0.0)
               - jnp.where(jg == gsrc_s[...], gw_s[...], 0.0))
        mg = _dot(Dg, WDg)
        mgh_s[...], mgl_s[...] = _split(mg)
        V_s[...] = jnp.full((nb, n_neur), -65.0, f32)
        m_s[...] = jnp.full((nb, n_neur), 0.0529, f32)
        h_s[...] = jnp.full((nb, n_neur), 0.5961, f32)
        n_s[...] = jnp.full((nb, n_neur), 0.3177, f32)

    Mch = mch_s[...]
    Mcl = mcl_s[...]
    Mgh = mgh_s[...]
    Mgl = mgl_s[...]

    V = V_s[...]
    m = m_s[...]
    h = h_s[...]
    n = n_s[...]
    for k in range(kt):
        I_t = iext_ref[:, 0, k, :]
        Q = jnp.exp((V + 40.0) * -0.1)
        P = jnp.exp((V + 65.0) * (-1.0 / 720.0))
        P2 = P * P
        P4 = P2 * P2
        P8 = P4 * P4
        P9 = P8 * P
        P16 = P8 * P8
        P32 = P16 * P16
        s_act = 1.0 / (1.0 + 54.598150033144236 * (Q * Q))
        g = _dot3(s_act, Mch, Mcl)
        I_chem = (0.0 - V) * g
        I_gap = -_dot3(V, Mgh, Mgl)
        I_Na = _G_NA * (m * m * m) * h * (V - _E_NA)
        n2 = n * n
        I_K = _G_K * (n2 * n2) * (V - _E_K)
        I_L = _G_L * (V - _E_L)
        dV = I_t + I_chem + I_gap - I_Na - I_K - I_L
        V_new = V + _DT * dV
        a_m = 0.1 * (V + 40.0) / (1.0 - Q + 1e-9)
        b_m = 4.0 * (P32 * P8)
        a_h = 0.07 * (P32 * P4)
        b_h = 1.0 / (1.0 + 1.6487212707001282 * Q)
        a_n = 0.01 * (V + 55.0) / (1.0 - 0.22313016014842982 * Q + 1e-9)
        b_n = 0.125 * P9
        m = jnp.clip(m + _DT * (a_m * (1.0 - m) - b_m * m), 0.0, 1.0)
        h = jnp.clip(h + _DT * (a_h * (1.0 - h) - b_h * h), 0.0, 1.0)
        n = jnp.clip(n + _DT * (a_n * (1.0 - n) - b_n * n), 0.0, 1.0)
        V = V_new
        vout_ref[:, 0, k, :] = V_new
    V_s[...] = V
    m_s[...] = m
    h_s[...] = h
    n_s[...] = n


def kernel(I_ext, chem_w, gap_w, chem_src, chem_dst, gap_src, gap_dst):
    nb, t, n_neur = I_ext.shape
    nc = chem_src.shape[0]
    ng = gap_src.shape[0]
    kt = next((k for k in (50, 25, 20, 10, 5, 2) if t % k == 0), 1)
    nbc = nb
    grid = (nb // nbc, t // kt)

    csrc_l = chem_src.reshape(1, nc).astype(jnp.int32)
    cdst_s = chem_dst.reshape(nc, 1).astype(jnp.int32)
    cw_s = chem_w.reshape(nc, 1)
    gsrc_l = gap_src.reshape(1, ng).astype(jnp.int32)
    gdst_l = gap_dst.reshape(1, ng).astype(jnp.int32)
    gsrc_s = gap_src.reshape(ng, 1).astype(jnp.int32)
    gdst_s = gap_dst.reshape(ng, 1).astype(jnp.int32)
    gw_s = gap_w.reshape(ng, 1)
    iext4 = I_ext.reshape(nb, t // kt, kt, n_neur)

    def body(csrc, cdst, cw, gsl, gdl, gss, gds, gw, iext, vout,
             V_s, m_s, h_s, n_s, mch_s, mcl_s, mgh_s, mgl_s):
        _hh_body(kt, n_neur, csrc, cdst, cw, gsl, gdl, gss, gds, gw,
                 iext, vout, V_s, m_s, h_s, n_s, mch_s, mcl_s, mgh_s, mgl_s)

    fixed = lambda b, i: (0, 0)
    out = pl.pallas_call(
        body,
        grid=grid,
        in_specs=[
            pl.BlockSpec((1, nc), fixed),
            pl.BlockSpec((nc, 1), fixed),
            pl.BlockSpec((nc, 1), fixed),
            pl.BlockSpec((1, ng), fixed),
            pl.BlockSpec((1, ng), fixed),
            pl.BlockSpec((ng, 1), fixed),
            pl.BlockSpec((ng, 1), fixed),
            pl.BlockSpec((ng, 1), fixed),
            pl.BlockSpec((nbc, 1, kt, n_neur), lambda b, i: (b, i, 0, 0)),
        ],
        out_specs=pl.BlockSpec((nbc, 1, kt, n_neur),
                               lambda b, i: (b, i, 0, 0)),
        out_shape=jax.ShapeDtypeStruct((nb, t // kt, kt, n_neur),
                                       jnp.float32),
        scratch_shapes=[
            pltpu.VMEM((nbc, n_neur), jnp.float32),
            pltpu.VMEM((nbc, n_neur), jnp.float32),
            pltpu.VMEM((nbc, n_neur), jnp.float32),
            pltpu.VMEM((nbc, n_neur), jnp.float32),
            pltpu.VMEM((n_neur, n_neur), jnp.bfloat16),
            pltpu.VMEM((n_neur, n_neur), jnp.bfloat16),
            pltpu.VMEM((n_neur, n_neur), jnp.bfloat16),
            pltpu.VMEM((n_neur, n_neur), jnp.bfloat16),
        ],
        compiler_params=pltpu.CompilerParams(
            dimension_semantics=("parallel", "arbitrary")),
    )(csrc_l, cdst_s, cw_s, gsrc_l, gdst_l, gsrc_s, gdst_s, gw_s, iext4)
    return out.reshape(nb, t, n_neur)


# speedup vs baseline: 1.2233x; 1.2233x over previous
import jax
import jax.numpy as jnp
from jax.experimental import pallas as pl
from jax.experimental.pallas import tpu as pltpu

_DT = 0.05
_G_NA = 120.0
_G_K = 36.0
_G_L = 0.3
_E_NA = 50.0
_E_K = -77.0
_E_L = -54.387

_HI = jax.lax.Precision.HIGHEST


def _dot(a, b, precision=_HI):
    return jax.lax.dot_general(a, b, (((1,), (0,)), ((), ())),
                               precision=precision,
                               preferred_element_type=jnp.float32)


def _split(x):
    hi = x.astype(jnp.bfloat16)
    lo = (x - hi.astype(jnp.float32)).astype(jnp.bfloat16)
    return hi, lo


def _dot3(x, m_hi, m_lo):
    x_hi, x_lo = _split(x)
    d = lambda a, b: _dot(a, b, precision=jax.lax.Precision.DEFAULT)
    return d(x_hi, m_hi) + d(x_hi, m_lo) + d(x_lo, m_hi)


def _hh_body(kt, n_neur, csrc_l, cdst_s, cw_s, gsrc_l, gdst_l, gsrc_s, gdst_s,
             gw_s, iext_ref, vout_ref, V_s, m_s, h_s, n_s,
             mch_s, mcl_s, mgh_s, mgl_s):
    nb = V_s.shape[0]
    nc = csrc_l.shape[1]
    ng = gsrc_l.shape[1]

    @pl.when(pl.program_id(1) == 0)
    def _init():
        f32 = jnp.float32
        ii = jax.lax.broadcasted_iota(jnp.int32, (n_neur, nc), 0)
        A = (ii == csrc_l[...]).astype(f32)
        jj = jax.lax.broadcasted_iota(jnp.int32, (nc, n_neur), 1)
        Wd = jnp.where(jj == cdst_s[...], cw_s[...], 0.0)
        mc = _dot(A, Wd)
        mch_s[...], mcl_s[...] = _split(mc)
        ig = jax.lax.broadcasted_iota(jnp.int32, (n_neur, ng), 0)
        Dg = ((ig == gdst_l[...]).astype(f32)
              - (ig == gsrc_l[...]).astype(f32))
        jg = jax.lax.broadcasted_iota(jnp.int32, (ng, n_neur), 1)
        WDg = (jnp.where(jg == gdst_s[...], gw_s[...], 0.0)
               - jnp.where(jg == gsrc_s[...], gw_s[...], 0.0))
        mg = _dot(Dg, WDg)
        mgh_s[...], mgl_s[...] = _split(mg)
        V_s[...] = jnp.full((nb, n_neur), -65.0, f32)
        m_s[...] = jnp.full((nb, n_neur), 0.0529, f32)
        h_s[...] = jnp.full((nb, n_neur), 0.5961, f32)
        n_s[...] = jnp.full((nb, n_neur), 0.3177, f32)

    Mch = mch_s[...]
    Mcl = mcl_s[...]
    Mgh = mgh_s[...]
    Mgl = mgl_s[...]

    V = V_s[...]
    m = m_s[...]
    h = h_s[...]
    n = n_s[...]
    for k in range(kt):
        I_t = iext_ref[:, 0, k, :]
        Q = jnp.exp((V + 40.0) * -0.1)
        P = jnp.exp((V + 65.0) * (-1.0 / 720.0))
        P2 = P * P
        P4 = P2 * P2
        P8 = P4 * P4
        P9 = P8 * P
        P16 = P8 * P8
        P32 = P16 * P16
        s_act = 1.0 / (1.0 + 54.598150033144236 * (Q * Q))
        g = _dot3(s_act, Mch, Mcl)
        I_chem = (0.0 - V) * g
        I_gap = -_dot3(V, Mgh, Mgl)
        I_Na = _G_NA * (m * m * m) * h * (V - _E_NA)
        n2 = n * n
        I_K = _G_K * (n2 * n2) * (V - _E_K)
        I_L = _G_L * (V - _E_L)
        dV = I_t + I_chem + I_gap - I_Na - I_K - I_L
        V_new = V + _DT * dV
        a_m = 0.1 * (V + 40.0) / (1.0 - Q + 1e-9)
        b_m = 4.0 * (P32 * P8)
        a_h = 0.07 * (P32 * P4)
        b_h = 1.0 / (1.0 + 1.6487212707001282 * Q)
        a_n = 0.01 * (V + 55.0) / (1.0 - 0.22313016014842982 * Q + 1e-9)
        b_n = 0.125 * P9
        m = jnp.clip(m + _DT * (a_m * (1.0 - m) - b_m * m), 0.0, 1.0)
        h = jnp.clip(h + _DT * (a_h * (1.0 - h) - b_h * h), 0.0, 1.0)
        n = jnp.clip(n + _DT * (a_n * (1.0 - n) - b_n * n), 0.0, 1.0)
        V = V_new
        vout_ref[:, 0, k, :] = V_new
    V_s[...] = V
    m_s[...] = m
    h_s[...] = h
    n_s[...] = n


def kernel(I_ext, chem_w, gap_w, chem_src, chem_dst, gap_src, gap_dst):
    nb, t, n_neur = I_ext.shape
    nc = chem_src.shape[0]
    ng = gap_src.shape[0]
    kt = next((k for k in (10, 20, 25, 5, 2) if t % k == 0), 1)
    nbc = nb
    grid = (nb // nbc, t // kt)

    csrc_l = chem_src.reshape(1, nc).astype(jnp.int32)
    cdst_s = chem_dst.reshape(nc, 1).astype(jnp.int32)
    cw_s = chem_w.reshape(nc, 1)
    gsrc_l = gap_src.reshape(1, ng).astype(jnp.int32)
    gdst_l = gap_dst.reshape(1, ng).astype(jnp.int32)
    gsrc_s = gap_src.reshape(ng, 1).astype(jnp.int32)
    gdst_s = gap_dst.reshape(ng, 1).astype(jnp.int32)
    gw_s = gap_w.reshape(ng, 1)
    iext4 = I_ext.reshape(nb, t // kt, kt, n_neur)

    def body(csrc, cdst, cw, gsl, gdl, gss, gds, gw, iext, vout,
             V_s, m_s, h_s, n_s, mch_s, mcl_s, mgh_s, mgl_s):
        _hh_body(kt, n_neur, csrc, cdst, cw, gsl, gdl, gss, gds, gw,
                 iext, vout, V_s, m_s, h_s, n_s, mch_s, mcl_s, mgh_s, mgl_s)

    fixed = lambda b, i: (0, 0)
    out = pl.pallas_call(
        body,
        grid=grid,
        in_specs=[
            pl.BlockSpec((1, nc), fixed),
            pl.BlockSpec((nc, 1), fixed),
            pl.BlockSpec((nc, 1), fixed),
            pl.BlockSpec((1, ng), fixed),
            pl.BlockSpec((1, ng), fixed),
            pl.BlockSpec((ng, 1), fixed),
            pl.BlockSpec((ng, 1), fixed),
            pl.BlockSpec((ng, 1), fixed),
            pl.BlockSpec((nbc, 1, kt, n_neur), lambda b, i: (b, i, 0, 0)),
        ],
        out_specs=pl.BlockSpec((nbc, 1, kt, n_neur),
                               lambda b, i: (b, i, 0, 0)),
        out_shape=jax.ShapeDtypeStruct((nb, t // kt, kt, n_neur),
                                       jnp.float32),
        scratch_shapes=[
            pltpu.VMEM((nbc, n_neur), jnp.float32),
            pltpu.VMEM((nbc, n_neur), jnp.float32),
            pltpu.VMEM((nbc, n_neur), jnp.float32),
            pltpu.VMEM((nbc, n_neur), jnp.float32),
            pltpu.VMEM((n_neur, n_neur), jnp.bfloat16),
            pltpu.VMEM((n_neur, n_neur), jnp.bfloat16),
            pltpu.VMEM((n_neur, n_neur), jnp.bfloat16),
            pltpu.VMEM((n_neur, n_neur), jnp.bfloat16),
        ],
        compiler_params=pltpu.CompilerParams(
            dimension_semantics=("parallel", "arbitrary")),
    )(csrc_l, cdst_s, cw_s, gsrc_l, gdst_l, gsrc_s, gdst_s, gw_s, iext4)
    return out.reshape(nb, t, n_neur)


# speedup vs baseline: 1.2791x; 1.0456x over previous
import jax
import jax.numpy as jnp
from jax.experimental import pallas as pl
from jax.experimental.pallas import tpu as pltpu

_DT = 0.05
_G_NA = 120.0
_G_K = 36.0
_G_L = 0.3
_E_NA = 50.0
_E_K = -77.0
_E_L = -54.387

_HI = jax.lax.Precision.HIGHEST


def _dot(a, b, precision=_HI):
    return jax.lax.dot_general(a, b, (((1,), (0,)), ((), ())),
                               precision=precision,
                               preferred_element_type=jnp.float32)


def _split(x):
    hi = x.astype(jnp.bfloat16)
    lo = (x - hi.astype(jnp.float32)).astype(jnp.bfloat16)
    return hi, lo


def _dot3(x, m_hi, m_lo):
    x_hi, x_lo = _split(x)
    d = lambda a, b: _dot(a, b, precision=jax.lax.Precision.DEFAULT)
    return d(x_hi, m_hi) + d(x_hi, m_lo) + d(x_lo, m_hi)


def _hh_body(kt, n_neur, csrc_l, cdst_s, cw_s, gsrc_l, gdst_l, gsrc_s, gdst_s,
             gw_s, iext_ref, vout_ref, V_s, m_s, h_s, n_s,
             mch_s, mcl_s, mgh_s, mgl_s):
    nb = V_s.shape[0]
    nc = csrc_l.shape[1]
    ng = gsrc_l.shape[1]

    @pl.when(pl.program_id(1) == 0)
    def _init():
        f32 = jnp.float32
        ii = jax.lax.broadcasted_iota(jnp.int32, (n_neur, nc), 0)
        A = (ii == csrc_l[...]).astype(f32)
        jj = jax.lax.broadcasted_iota(jnp.int32, (nc, n_neur), 1)
        Wd = jnp.where(jj == cdst_s[...], cw_s[...], 0.0)
        mc = _dot(A, Wd)
        mch_s[...], mcl_s[...] = _split(mc)
        ig = jax.lax.broadcasted_iota(jnp.int32, (n_neur, ng), 0)
        Dg = ((ig == gdst_l[...]).astype(f32)
              - (ig == gsrc_l[...]).astype(f32))
        jg = jax.lax.broadcasted_iota(jnp.int32, (ng, n_neur), 1)
        WDg = (jnp.where(jg == gdst_s[...], gw_s[...], 0.0)
               - jnp.where(jg == gsrc_s[...], gw_s[...], 0.0))
        mg = _dot(Dg, WDg)
        mgh_s[...], mgl_s[...] = _split(mg)
        V_s[...] = jnp.full((nb, n_neur), -65.0, f32)
        m_s[...] = jnp.full((nb, n_neur), 0.0529, f32)
        h_s[...] = jnp.full((nb, n_neur), 0.5961, f32)
        n_s[...] = jnp.full((nb, n_neur), 0.3177, f32)

    Mch = mch_s[...]
    Mcl = mcl_s[...]
    Mgh = mgh_s[...]
    Mgl = mgl_s[...]

    V = V_s[...]
    m = m_s[...]
    h = h_s[...]
    n = n_s[...]
    for k in range(kt):
        I_t = iext_ref[:, 0, k, :]
        Q = jnp.exp((V + 40.0) * -0.1)
        P = jnp.exp((V + 65.0) * (-1.0 / 720.0))
        P2 = P * P
        P4 = P2 * P2
        P8 = P4 * P4
        P9 = P8 * P
        P16 = P8 * P8
        P32 = P16 * P16
        s_act = 1.0 / (1.0 + 54.598150033144236 * (Q * Q))
        g = _dot3(s_act, Mch, Mcl)
        I_chem = (0.0 - V) * g
        I_gap = -_dot3(V, Mgh, Mgl)
        I_Na = _G_NA * (m * m * m) * h * (V - _E_NA)
        n2 = n * n
        I_K = _G_K * (n2 * n2) * (V - _E_K)
        I_L = _G_L * (V - _E_L)
        dV = I_t + I_chem + I_gap - I_Na - I_K - I_L
        V_new = V + _DT * dV
        a_m = 0.1 * (V + 40.0) / (1.0 - Q + 1e-9)
        b_m = 4.0 * (P32 * P8)
        a_h = 0.07 * (P32 * P4)
        b_h = 1.0 / (1.0 + 1.6487212707001282 * Q)
        a_n = 0.01 * (V + 55.0) / (1.0 - 0.22313016014842982 * Q + 1e-9)
        b_n = 0.125 * P9
        m = jnp.clip(m + _DT * (a_m * (1.0 - m) - b_m * m), 0.0, 1.0)
        h = jnp.clip(h + _DT * (a_h * (1.0 - h) - b_h * h), 0.0, 1.0)
        n = jnp.clip(n + _DT * (a_n * (1.0 - n) - b_n * n), 0.0, 1.0)
        V = V_new
        vout_ref[:, 0, k, :] = V_new
    V_s[...] = V
    m_s[...] = m
    h_s[...] = h
    n_s[...] = n


def kernel(I_ext, chem_w, gap_w, chem_src, chem_dst, gap_src, gap_dst):
    nb, t, n_neur = I_ext.shape
    nc = chem_src.shape[0]
    ng = gap_src.shape[0]
    kt = next((k for k in (5, 10, 20, 25, 2) if t % k == 0), 1)
    nbc = nb
    grid = (nb // nbc, t // kt)

    csrc_l = chem_src.reshape(1, nc).astype(jnp.int32)
    cdst_s = chem_dst.reshape(nc, 1).astype(jnp.int32)
    cw_s = chem_w.reshape(nc, 1)
    gsrc_l = gap_src.reshape(1, ng).astype(jnp.int32)
    gdst_l = gap_dst.reshape(1, ng).astype(jnp.int32)
    gsrc_s = gap_src.reshape(ng, 1).astype(jnp.int32)
    gdst_s = gap_dst.reshape(ng, 1).astype(jnp.int32)
    gw_s = gap_w.reshape(ng, 1)
    iext4 = I_ext.reshape(nb, t // kt, kt, n_neur)

    def body(csrc, cdst, cw, gsl, gdl, gss, gds, gw, iext, vout,
             V_s, m_s, h_s, n_s, mch_s, mcl_s, mgh_s, mgl_s):
        _hh_body(kt, n_neur, csrc, cdst, cw, gsl, gdl, gss, gds, gw,
                 iext, vout, V_s, m_s, h_s, n_s, mch_s, mcl_s, mgh_s, mgl_s)

    fixed = lambda b, i: (0, 0)
    out = pl.pallas_call(
        body,
        grid=grid,
        in_specs=[
            pl.BlockSpec((1, nc), fixed),
            pl.BlockSpec((nc, 1), fixed),
            pl.BlockSpec((nc, 1), fixed),
            pl.BlockSpec((1, ng), fixed),
            pl.BlockSpec((1, ng), fixed),
            pl.BlockSpec((ng, 1), fixed),
            pl.BlockSpec((ng, 1), fixed),
            pl.BlockSpec((ng, 1), fixed),
            pl.BlockSpec((nbc, 1, kt, n_neur), lambda b, i: (b, i, 0, 0)),
        ],
        out_specs=pl.BlockSpec((nbc, 1, kt, n_neur),
                               lambda b, i: (b, i, 0, 0)),
        out_shape=jax.ShapeDtypeStruct((nb, t // kt, kt, n_neur),
                                       jnp.float32),
        scratch_shapes=[
            pltpu.VMEM((nbc, n_neur), jnp.float32),
            pltpu.VMEM((nbc, n_neur), jnp.float32),
            pltpu.VMEM((nbc, n_neur), jnp.float32),
            pltpu.VMEM((nbc, n_neur), jnp.float32),
            pltpu.VMEM((n_neur, n_neur), jnp.bfloat16),
            pltpu.VMEM((n_neur, n_neur), jnp.bfloat16),
            pltpu.VMEM((n_neur, n_neur), jnp.bfloat16),
            pltpu.VMEM((n_neur, n_neur), jnp.bfloat16),
        ],
        compiler_params=pltpu.CompilerParams(
            dimension_semantics=("parallel", "arbitrary")),
    )(csrc_l, cdst_s, cw_s, gsrc_l, gdst_l, gsrc_s, gdst_s, gw_s, iext4)
    return out.reshape(nb, t, n_neur)
